# baseline (speedup 1.0000x reference)
import functools

import jax
import jax.numpy as jnp
from jax import lax
from jax.experimental import pallas as pl
from jax.experimental.pallas import tpu as pltpu
from jax.experimental.pallas import tpu_sc as plsc

NUM_FIELDS = 26
VOCAB = 100000
EMBED_DIM = 32
BATCH = 16384

NUM_CORES = 2
NUM_SUBCORES = 16
NW = NUM_CORES * NUM_SUBCORES
LANES = 16

NV = 4
TAIL = VOCAB % 128
TAIL_PAD = 128
VMAIN = VOCAB - TAIL
CS = [25088, 25088, 25088, 24704]
OFF = [0, 25088, 50176, 75264]
PASS_SZ = [25088, 25088, 25088, 24704 + TAIL]
CMAX = 25088
HALF_B = BATCH // 2


def _make_kernel(interpret=False):
    mesh = plsc.VectorSubcoreMesh(
        core_axis_name="c", subcore_axis_name="s",
        num_cores=NUM_CORES, num_subcores=NUM_SUBCORES)

    @functools.partial(
        pl.kernel,
        mesh=mesh,
        out_type=jax.ShapeDtypeStruct((NUM_FIELDS, EMBED_DIM, BATCH),
                                      jnp.float32),
        scratch_types=[
            pltpu.VMEM((CMAX,), jnp.float32),
            pltpu.VMEM((CMAX,), jnp.float32),
            pltpu.VMEM((BATCH,), jnp.int32),
            pltpu.VMEM((BATCH,), jnp.int32),
            pltpu.VMEM((BATCH,), jnp.float32),
            pltpu.VMEM((BATCH,), jnp.float32),
            pltpu.SemaphoreType.DMA,
            pltpu.SemaphoreType.DMA,
            pltpu.SemaphoreType.DMA,
            pltpu.SemaphoreType.DMA,
            pltpu.SemaphoreType.DMA,
            pltpu.SemaphoreType.DMA,
        ],
        compiler_params=pltpu.CompilerParams(use_tc_tiling_on_sc=True,
                                             needs_layout_passes=False),
        interpret=interpret,
    )
    def gather_kernel(tab_hbm, idx_hbm, tail_hbm, out_hbm, tv0, tv1,
                      xv0, xv1, ov0, ov1, ts0, ts1, xs0, xs1, ws0, ws1):
        wid = lax.axis_index("s") * NUM_CORES + lax.axis_index("c")
        tvs, tss = (tv0, tv1), (ts0, ts1)
        xvs, xss = (xv0, xv1), (xs0, xs1)
        ovs, wss = (ov0, ov1), (ws0, ws1)
        iota = lax.iota(jnp.int32, LANES)

        def tab_chunk(f, c):
            return tab_hbm.at[f, wid, pl.ds(OFF[c], CS[c])]

        def tv_slice(slot, c):
            return tvs[slot].at[pl.ds(0, CS[c])]

        def issue_chunk(f, c, slot):
            pltpu.async_copy(tab_chunk(f, c), tv_slice(slot, c),
                             tss[slot])
            if c == NV - 1:
                pltpu.async_copy(tail_hbm.at[f, wid],
                                 tvs[slot].at[pl.ds(CS[c], TAIL_PAD)],
                                 tss[slot])

        def wait_chunk(f, c, slot):
            pltpu.make_async_copy(tab_chunk(f, c), tv_slice(slot, c),
                                  tss[slot]).wait()
            if c == NV - 1:
                pltpu.make_async_copy(tail_hbm.at[f, wid],
                                      tvs[slot].at[pl.ds(CS[c], TAIL_PAD)],
                                      tss[slot]).wait()

        pltpu.async_copy(idx_hbm.at[0], xvs[0], xss[0])
        issue_chunk(0, 0, 0)
        issue_chunk(0, 1, 1)

        def chunk_pass(f, c, slot, xv, ov):
            lo = OFF[c]
            size = PASS_SZ[c]
            tv = tvs[slot]
            for b in range(2):
                boff = b * HALF_B

                @pl.loop(0, HALF_B // LANES, unroll=8)
                def inner(i):
                    idx = xv[pl.ds(boff + i * LANES, LANES)]
                    rel = plsc.bitcast(idx - lo, jnp.uint32)
                    mask = rel < jnp.uint32(size)
                    relc = plsc.bitcast(
                        jnp.minimum(rel, jnp.uint32(size - 1)), jnp.int32)
                    vals = plsc.load_gather(tv, [relc])
                    pos = iota + (boff + i * LANES)
                    plsc.store_scatter(ov, [pos], vals, mask=mask)

        def do_field(g0, b):
            f = g0 + b
            xv, xs = xvs[b], xss[b]
            ov, ws = ovs[b], wss[b]

            pltpu.make_async_copy(idx_hbm.at[f], xv, xs).wait()

            @pl.when(f + 1 < NUM_FIELDS)
            def _():
                pltpu.async_copy(idx_hbm.at[f + 1], xvs[1 - b],
                                 xss[1 - b])

            @pl.when(g0 >= 2 - b)
            def _():
                pltpu.make_async_copy(ov, out_hbm.at[f - 2, wid],
                                      ws).wait()

            for c in range(NV):
                slot = c % 2
                wait_chunk(f, c, slot)
                chunk_pass(f, c, slot, xv, ov)
                c2 = c + 2
                if c2 < NV:
                    issue_chunk(f, c2, slot)
                else:

                    @pl.when(f + 1 < NUM_FIELDS)
                    def _():
                        issue_chunk(f + 1, c2 - NV, slot)

            pltpu.async_copy(ov, out_hbm.at[f, wid], ws)

        @pl.loop(0, NUM_FIELDS, step=2)
        def fields(g0):
            for b in range(2):
                do_field(g0, b)

        pltpu.make_async_copy(ovs[0], out_hbm.at[NUM_FIELDS - 2, wid],
                              wss[0]).wait()
        pltpu.make_async_copy(ovs[1], out_hbm.at[NUM_FIELDS - 1, wid],
                              wss[1]).wait()

    return gather_kernel


_gather = _make_kernel()


@jax.jit
def kernel(x, tables):
    tables_t = jnp.transpose(tables, (0, 2, 1))
    x_t = jnp.transpose(x.astype(jnp.int32), (1, 0))
    tail_t = jnp.pad(tables_t[:, :, VMAIN:],
                     ((0, 0), (0, 0), (0, TAIL_PAD - TAIL)))
    out_t = _gather(tables_t, x_t, tail_t)
    return jnp.transpose(out_t, (2, 0, 1))


# speedup vs baseline: 4.3588x; 4.3588x over previous
import functools

import jax
import jax.numpy as jnp
from jax import lax
from jax.experimental import pallas as pl
from jax.experimental.pallas import tpu as pltpu
from jax.experimental.pallas import tpu_sc as plsc

NUM_FIELDS = 26
VOCAB = 100000
EMBED_DIM = 32
BATCH = 16384

NUM_CORES = 2
NUM_SUBCORES = 16
NW = NUM_CORES * NUM_SUBCORES
LANES = 16

NV = 4
TAIL = VOCAB % 128
TAIL_PAD = 128
VMAIN = VOCAB - TAIL
CS = [25088, 25088, 25088, 24704]
OFF = [0, 25088, 50176, 75264]
PASS_SZ = [25088, 25088, 25088, 24704 + TAIL]
CMAX = 25088
HALF_B = BATCH // 2


def _make_kernel(interpret=False):
    mesh = plsc.VectorSubcoreMesh(
        core_axis_name="c", subcore_axis_name="s",
        num_cores=NUM_CORES, num_subcores=NUM_SUBCORES)

    @functools.partial(
        pl.kernel,
        mesh=mesh,
        out_type=jax.ShapeDtypeStruct((NUM_FIELDS, EMBED_DIM, BATCH),
                                      jnp.float32),
        scratch_types=[
            pltpu.VMEM((CMAX,), jnp.float32),
            pltpu.VMEM((CMAX,), jnp.float32),
            pltpu.VMEM((BATCH,), jnp.int32),
            pltpu.VMEM((BATCH,), jnp.int32),
            pltpu.VMEM((BATCH,), jnp.float32),
            pltpu.VMEM((BATCH,), jnp.float32),
            pltpu.SemaphoreType.DMA,
            pltpu.SemaphoreType.DMA,
            pltpu.SemaphoreType.DMA,
            pltpu.SemaphoreType.DMA,
            pltpu.SemaphoreType.DMA,
            pltpu.SemaphoreType.DMA,
        ],
        compiler_params=pltpu.CompilerParams(use_tc_tiling_on_sc=True,
                                             needs_layout_passes=False),
        interpret=interpret,
    )
    def gather_kernel(tab_hbm, idx_hbm, tail_hbm, out_hbm, tv0, tv1,
                      xv0, xv1, ov0, ov1, ts0, ts1, xs0, xs1, ws0, ws1):
        wid = lax.axis_index("s") * NUM_CORES + lax.axis_index("c")
        tvs, tss = (tv0, tv1), (ts0, ts1)
        xvs, xss = (xv0, xv1), (xs0, xs1)
        ovs, wss = (ov0, ov1), (ws0, ws1)
        iota = lax.iota(jnp.int32, LANES)

        def tab_chunk(f, c):
            return tab_hbm.at[f, wid, pl.ds(OFF[c], CS[c])]

        def tv_slice(slot, c):
            return tvs[slot].at[pl.ds(0, CS[c])]

        def issue_chunk(f, c, slot):
            pltpu.async_copy(tab_chunk(f, c), tv_slice(slot, c),
                             tss[slot])
            if c == NV - 1:
                pltpu.async_copy(tail_hbm.at[f, wid],
                                 tvs[slot].at[pl.ds(CS[c], TAIL_PAD)],
                                 tss[slot])

        def wait_chunk(f, c, slot):
            pltpu.make_async_copy(tab_chunk(f, c), tv_slice(slot, c),
                                  tss[slot]).wait()
            if c == NV - 1:
                pltpu.make_async_copy(tail_hbm.at[f, wid],
                                      tvs[slot].at[pl.ds(CS[c], TAIL_PAD)],
                                      tss[slot]).wait()

        pltpu.async_copy(idx_hbm.at[0], xvs[0], xss[0])
        issue_chunk(0, 0, 0)
        issue_chunk(0, 1, 1)

        def chunk_pass(f, c, slot, xv, ov):
            lo = OFF[c]
            size = PASS_SZ[c]
            tv = tvs[slot]
            for b in range(2):
                boff = b * HALF_B

                @plsc.parallel_loop(0, HALF_B // LANES, unroll=8)
                def inner(i):
                    idx = xv[pl.ds(boff + i * LANES, LANES)]
                    rel = plsc.bitcast(idx - lo, jnp.uint32)
                    mask = rel < jnp.uint32(size)
                    relc = plsc.bitcast(
                        jnp.minimum(rel, jnp.uint32(size - 1)), jnp.int32)
                    vals = plsc.load_gather(tv, [relc])
                    pos = iota + (boff + i * LANES)
                    plsc.store_scatter(ov, [pos], vals, mask=mask)

        def do_field(g0, b):
            f = g0 + b
            xv, xs = xvs[b], xss[b]
            ov, ws = ovs[b], wss[b]

            pltpu.make_async_copy(idx_hbm.at[f], xv, xs).wait()

            @pl.when(f + 1 < NUM_FIELDS)
            def _():
                pltpu.async_copy(idx_hbm.at[f + 1], xvs[1 - b],
                                 xss[1 - b])

            @pl.when(g0 >= 2 - b)
            def _():
                pltpu.make_async_copy(ov, out_hbm.at[f - 2, wid],
                                      ws).wait()

            for c in range(NV):
                slot = c % 2
                wait_chunk(f, c, slot)
                chunk_pass(f, c, slot, xv, ov)
                c2 = c + 2
                if c2 < NV:
                    issue_chunk(f, c2, slot)
                else:

                    @pl.when(f + 1 < NUM_FIELDS)
                    def _():
                        issue_chunk(f + 1, c2 - NV, slot)

            pltpu.async_copy(ov, out_hbm.at[f, wid], ws)

        @pl.loop(0, NUM_FIELDS, step=2)
        def fields(g0):
            for b in range(2):
                do_field(g0, b)

        pltpu.make_async_copy(ovs[0], out_hbm.at[NUM_FIELDS - 2, wid],
                              wss[0]).wait()
        pltpu.make_async_copy(ovs[1], out_hbm.at[NUM_FIELDS - 1, wid],
                              wss[1]).wait()

    return gather_kernel


_gather = _make_kernel()


@jax.jit
def kernel(x, tables):
    tables_t = jnp.transpose(tables, (0, 2, 1))
    x_t = jnp.transpose(x.astype(jnp.int32), (1, 0))
    tail_t = jnp.pad(tables_t[:, :, VMAIN:],
                     ((0, 0), (0, 0), (0, TAIL_PAD - TAIL)))
    out_t = _gather(tables_t, x_t, tail_t)
    return jnp.transpose(out_t, (2, 0, 1))


# speedup vs baseline: 4.9709x; 1.1404x over previous
import functools

import jax
import jax.numpy as jnp
from jax import lax
from jax.experimental import pallas as pl
from jax.experimental.pallas import tpu as pltpu
from jax.experimental.pallas import tpu_sc as plsc

NUM_FIELDS = 26
VOCAB = 100000
EMBED_DIM = 32
BATCH = 16384

NUM_CORES = 2
NUM_SUBCORES = 16
NW = NUM_CORES * NUM_SUBCORES
LANES = 16

NV = 3
TAIL = VOCAB % 128
TAIL_PAD = 128
VMAIN = VOCAB - TAIL
CS = [33408, 33408, 33152]
OFF = [0, 33408, 66816]
PASS_SZ = [33408, 33408, 33152 + TAIL]
CMAX = 33408


def _make_kernel(interpret=False):
    mesh = plsc.VectorSubcoreMesh(
        core_axis_name="c", subcore_axis_name="s",
        num_cores=NUM_CORES, num_subcores=NUM_SUBCORES)

    @functools.partial(
        pl.kernel,
        mesh=mesh,
        out_type=jax.ShapeDtypeStruct((NUM_FIELDS, EMBED_DIM, BATCH),
                                      jnp.float32),
        scratch_types=[
            pltpu.VMEM((CMAX,), jnp.float32),
            pltpu.VMEM((CMAX,), jnp.float32),
            pltpu.VMEM((BATCH,), jnp.int32),
            pltpu.VMEM((BATCH,), jnp.int32),
            pltpu.VMEM((BATCH,), jnp.float32),
            pltpu.SemaphoreType.DMA,
            pltpu.SemaphoreType.DMA,
            pltpu.SemaphoreType.DMA,
            pltpu.SemaphoreType.DMA,
            pltpu.SemaphoreType.DMA,
        ],
        compiler_params=pltpu.CompilerParams(use_tc_tiling_on_sc=True,
                                             needs_layout_passes=False),
        interpret=interpret,
    )
    def gather_kernel(tab_hbm, idx_hbm, tail_hbm, out_hbm, tv0, tv1,
                      xv0, xv1, ov, ts0, ts1, xs0, xs1, ws):
        wid = lax.axis_index("s") * NUM_CORES + lax.axis_index("c")
        tvs, tss = (tv0, tv1), (ts0, ts1)
        xvs, xss = (xv0, xv1), (xs0, xs1)
        iota = lax.iota(jnp.int32, LANES)

        def tab_chunk(f, c):
            return tab_hbm.at[f, wid, pl.ds(OFF[c], CS[c])]

        def tv_slice(slot, c):
            return tvs[slot].at[pl.ds(0, CS[c])]

        def issue_chunk(f, c, slot):
            pltpu.async_copy(tab_chunk(f, c), tv_slice(slot, c),
                             tss[slot])
            if c == NV - 1:
                pltpu.async_copy(tail_hbm.at[f, wid],
                                 tvs[slot].at[pl.ds(CS[c], TAIL_PAD)],
                                 tss[slot])

        def wait_chunk(f, c, slot):
            pltpu.make_async_copy(tab_chunk(f, c), tv_slice(slot, c),
                                  tss[slot]).wait()
            if c == NV - 1:
                pltpu.make_async_copy(tail_hbm.at[f, wid],
                                      tvs[slot].at[pl.ds(CS[c], TAIL_PAD)],
                                      tss[slot]).wait()

        pltpu.async_copy(idx_hbm.at[0], xvs[0], xss[0])
        issue_chunk(0, 0, 0)
        issue_chunk(0, 1, 1)

        def chunk_pass(f, c, slot, xv, ov):
            lo = OFF[c]
            size = PASS_SZ[c]
            tv = tvs[slot]

            @plsc.parallel_loop(0, BATCH // LANES, unroll=8)
            def inner(i):
                idx = xv[pl.ds(i * LANES, LANES)]
                rel = plsc.bitcast(idx - lo, jnp.uint32)
                mask = rel < jnp.uint32(size)
                relc = plsc.bitcast(
                    jnp.minimum(rel, jnp.uint32(size - 1)), jnp.int32)
                vals = plsc.load_gather(tv, [relc])
                pos = iota + i * LANES
                plsc.store_scatter(ov, [pos], vals, mask=mask)

        def do_field(g0, b):
            f = g0 + b
            xv, xs = xvs[b], xss[b]

            pltpu.make_async_copy(idx_hbm.at[f], xv, xs).wait()

            @pl.when(f + 1 < NUM_FIELDS)
            def _():
                pltpu.async_copy(idx_hbm.at[f + 1], xvs[1 - b],
                                 xss[1 - b])

            @pl.when(g0 + b >= 1)
            def _():
                pltpu.make_async_copy(ov, out_hbm.at[f - 1, wid],
                                      ws).wait()

            for c in range(NV):
                slot = (b * NV + c) % 2
                wait_chunk(f, c, slot)
                chunk_pass(f, c, slot, xv, ov)
                c2 = c + 2
                if c2 < NV:
                    issue_chunk(f, c2, slot)
                else:

                    @pl.when(f + 1 < NUM_FIELDS)
                    def _():
                        issue_chunk(f + 1, c2 - NV, slot)

            pltpu.async_copy(ov, out_hbm.at[f, wid], ws)

        @pl.loop(0, NUM_FIELDS, step=2)
        def fields(g0):
            for b in range(2):
                do_field(g0, b)

        pltpu.make_async_copy(ov, out_hbm.at[NUM_FIELDS - 1, wid],
                              ws).wait()

    return gather_kernel


_gather = _make_kernel()


@jax.jit
def kernel(x, tables):
    tables_t = jnp.transpose(tables, (0, 2, 1))
    x_t = jnp.transpose(x.astype(jnp.int32), (1, 0))
    tail_t = jnp.pad(tables_t[:, :, VMAIN:],
                     ((0, 0), (0, 0), (0, TAIL_PAD - TAIL)))
    out_t = _gather(tables_t, x_t, tail_t)
    return jnp.transpose(out_t, (2, 0, 1))


# speedup vs baseline: 4.9839x; 1.0026x over previous
import functools

import jax
import jax.numpy as jnp
from jax import lax
from jax.experimental import pallas as pl
from jax.experimental.pallas import tpu as pltpu
from jax.experimental.pallas import tpu_sc as plsc

NUM_FIELDS = 26
VOCAB = 100000
EMBED_DIM = 32
BATCH = 16384

NUM_CORES = 2
NUM_SUBCORES = 16
NW = NUM_CORES * NUM_SUBCORES
LANES = 16

NV = 3
TAIL = VOCAB % 128
TAIL_PAD = 128
VMAIN = VOCAB - TAIL
CS = [33408, 33408, 33152]
OFF = [0, 33408, 66816]
PASS_SZ = [33408, 33408, 33152 + TAIL]
CMAX = 33408


def _make_kernel(interpret=False):
    mesh = plsc.VectorSubcoreMesh(
        core_axis_name="c", subcore_axis_name="s",
        num_cores=NUM_CORES, num_subcores=NUM_SUBCORES)

    @functools.partial(
        pl.kernel,
        mesh=mesh,
        out_type=jax.ShapeDtypeStruct((NUM_FIELDS, EMBED_DIM, BATCH),
                                      jnp.float32),
        scratch_types=[
            pltpu.VMEM((CMAX,), jnp.float32),
            pltpu.VMEM((CMAX,), jnp.float32),
            pltpu.VMEM((BATCH,), jnp.int32),
            pltpu.VMEM((BATCH,), jnp.int32),
            pltpu.VMEM((BATCH,), jnp.float32),
            pltpu.SemaphoreType.DMA,
            pltpu.SemaphoreType.DMA,
            pltpu.SemaphoreType.DMA,
            pltpu.SemaphoreType.DMA,
            pltpu.SemaphoreType.DMA,
        ],
        compiler_params=pltpu.CompilerParams(use_tc_tiling_on_sc=True,
                                             needs_layout_passes=False),
        interpret=interpret,
    )
    def gather_kernel(tab_hbm, idx_hbm, tail_hbm, out_hbm, tv0, tv1,
                      xv0, xv1, ov, ts0, ts1, xs0, xs1, ws):
        wid = lax.axis_index("s") * NUM_CORES + lax.axis_index("c")
        tvs, tss = (tv0, tv1), (ts0, ts1)
        xvs, xss = (xv0, xv1), (xs0, xs1)
        iota = lax.iota(jnp.int32, LANES)

        def tab_chunk(f, c):
            return tab_hbm.at[f, wid, pl.ds(OFF[c], CS[c])]

        def tv_slice(slot, c):
            return tvs[slot].at[pl.ds(0, CS[c])]

        def issue_chunk(f, c, slot):
            pltpu.async_copy(tab_chunk(f, c), tv_slice(slot, c),
                             tss[slot])
            if c == NV - 1:
                pltpu.async_copy(tail_hbm.at[f, wid],
                                 tvs[slot].at[pl.ds(CS[c], TAIL_PAD)],
                                 tss[slot])

        def wait_chunk(f, c, slot):
            pltpu.make_async_copy(tab_chunk(f, c), tv_slice(slot, c),
                                  tss[slot]).wait()
            if c == NV - 1:
                pltpu.make_async_copy(tail_hbm.at[f, wid],
                                      tvs[slot].at[pl.ds(CS[c], TAIL_PAD)],
                                      tss[slot]).wait()

        pltpu.async_copy(idx_hbm.at[0], xvs[0], xss[0])
        issue_chunk(0, 0, 0)
        issue_chunk(0, 1, 1)

        def chunk_pass(f, c, slot, xv, ov):
            lo = OFF[c]
            size = PASS_SZ[c]
            tv = tvs[slot]

            @plsc.parallel_loop(0, BATCH // LANES, unroll=16)
            def inner(i):
                idx = xv[pl.ds(i * LANES, LANES)]
                rel = plsc.bitcast(idx - lo, jnp.uint32)
                mask = rel < jnp.uint32(size)
                relc = plsc.bitcast(
                    jnp.minimum(rel, jnp.uint32(size - 1)), jnp.int32)
                vals = plsc.load_gather(tv, [relc])
                pos = iota + i * LANES
                plsc.store_scatter(ov, [pos], vals, mask=mask)

        def do_field(g0, b):
            f = g0 + b
            xv, xs = xvs[b], xss[b]

            pltpu.make_async_copy(idx_hbm.at[f], xv, xs).wait()

            @pl.when(f + 1 < NUM_FIELDS)
            def _():
                pltpu.async_copy(idx_hbm.at[f + 1], xvs[1 - b],
                                 xss[1 - b])

            @pl.when(g0 + b >= 1)
            def _():
                pltpu.make_async_copy(ov, out_hbm.at[f - 1, wid],
                                      ws).wait()

            for c in range(NV):
                slot = (b * NV + c) % 2
                wait_chunk(f, c, slot)
                chunk_pass(f, c, slot, xv, ov)
                c2 = c + 2
                if c2 < NV:
                    issue_chunk(f, c2, slot)
                else:

                    @pl.when(f + 1 < NUM_FIELDS)
                    def _():
                        issue_chunk(f + 1, c2 - NV, slot)

            pltpu.async_copy(ov, out_hbm.at[f, wid], ws)

        @pl.loop(0, NUM_FIELDS, step=2)
        def fields(g0):
            for b in range(2):
                do_field(g0, b)

        pltpu.make_async_copy(ov, out_hbm.at[NUM_FIELDS - 1, wid],
                              ws).wait()

    return gather_kernel


_gather = _make_kernel()


@jax.jit
def kernel(x, tables):
    tables_t = jnp.transpose(tables, (0, 2, 1))
    x_t = jnp.transpose(x.astype(jnp.int32), (1, 0))
    tail_t = jnp.pad(tables_t[:, :, VMAIN:],
                     ((0, 0), (0, 0), (0, TAIL_PAD - TAIL)))
    out_t = _gather(tables_t, x_t, tail_t)
    return jnp.transpose(out_t, (2, 0, 1))
